# Initial kernel scaffold; baseline (speedup 1.0000x reference)
import jax
import jax.numpy as jnp
from jax.experimental import pallas as pl


def kernel(out_1, out_2, batch_size, temperature):
    raise NotImplementedError("write your pallas kernel here")


# speedup vs baseline: 28.4446x; 28.4446x over previous
import functools

import jax
import jax.numpy as jnp
from jax.experimental import pallas as pl
from jax.experimental.pallas import tpu as pltpu

_MOMENTUM = 0.99
_REGULAR = 0.1
_T0_SCALE = 1.0 - _MOMENTUM


def _tl_kernel(o1_ref, o2_ref, tl_ref, t0_ref):
    p = jnp.sum(o1_ref[:, :] * o2_ref[:, :], axis=1, keepdims=True)
    p = jnp.clip(p, -1.0, 1.0)
    b = p.shape[0]
    tl_ref[0:b, :] = p
    tl_ref[b:2 * b, :] = p
    t0_ref[0, 0] = _T0_SCALE * jnp.mean(p)


def _loss_kernel(temp_ref, xr_ref, xc_ref, tl_ref, t0_ref, out_ref,
                 sacc_ref, reacc_ref, lacc_ref, *, nbi, nbj, n):
    i = pl.program_id(0)
    j = pl.program_id(1)
    rb = xr_ref.shape[0]
    cb = xc_ref.shape[0]
    t0 = t0_ref[0, 0]
    inv_t = 1.0 / temp_ref[0, 0]

    @pl.when(jnp.logical_and(i == 0, j == 0))
    def _():
        lacc_ref[0, 0] = 0.0

    @pl.when(j == 0)
    def _():
        sacc_ref[:, :] = jnp.zeros_like(sacc_ref)
        reacc_ref[:, :] = jnp.zeros_like(reacc_ref)

    s = jax.lax.dot_general(
        xr_ref[:, :], xc_ref[:, :], (((1,), (1,)), ((), ())),
        preferred_element_type=jnp.float32,
        precision=jax.lax.Precision.HIGHEST)
    v = jnp.clip(s, -1.0, 1.0)
    tl = tl_ref[:, :]
    m = v > tl
    diag = (jax.lax.broadcasted_iota(jnp.int32, (rb, cb), 0)
            + (i * rb - j * cb)) == jax.lax.broadcasted_iota(
                jnp.int32, (rb, cb), 1)
    mod = jnp.where(m, v * (t0 + v), v)
    e = jnp.where(diag, 0.0, jnp.exp(mod * inv_t))
    sacc_ref[:, :] += jnp.sum(e, axis=1, keepdims=True)
    w = v + t0
    rw = jnp.where(jnp.logical_and(m, w > 1.0) & jnp.logical_not(diag),
                   w, 0.0)
    reacc_ref[:, :] += jnp.sum(rw * rw, axis=1, keepdims=True)

    @pl.when(j == nbj - 1)
    def _():
        ce_part = jnp.sum(jnp.log(sacc_ref[:, :]) - tl_ref[:, :] * inv_t)
        lacc_ref[0, 0] += (ce_part / n
                           + (0.5 * _REGULAR) * jnp.sum(reacc_ref[:, :]))

        @pl.when(i == nbi - 1)
        def _():
            out_ref[0, 0] = lacc_ref[0, 0]


def kernel(out_1, out_2, batch_size, temperature):
    del batch_size
    b, d = out_1.shape
    n = 2 * b
    x = jnp.concatenate([out_1, out_2], axis=0)
    temp = jnp.asarray(temperature, jnp.float32).reshape(1, 1)

    tl, t0 = pl.pallas_call(
        _tl_kernel,
        out_shape=(
            jax.ShapeDtypeStruct((n, 1), jnp.float32),
            jax.ShapeDtypeStruct((1, 1), jnp.float32),
        ),
        out_specs=(
            pl.BlockSpec(memory_space=pltpu.VMEM),
            pl.BlockSpec(memory_space=pltpu.SMEM),
        ),
        in_specs=(
            pl.BlockSpec(memory_space=pltpu.VMEM),
            pl.BlockSpec(memory_space=pltpu.VMEM),
        ),
    )(out_1, out_2)

    rb = 1024
    cb = 1024
    nbi = n // rb
    nbj = n // cb

    body = functools.partial(_loss_kernel, nbi=nbi, nbj=nbj, n=float(n))
    loss = pl.pallas_call(
        body,
        grid=(nbi, nbj),
        in_specs=[
            pl.BlockSpec(memory_space=pltpu.SMEM),
            pl.BlockSpec((rb, d), lambda i, j: (i, 0)),
            pl.BlockSpec((cb, d), lambda i, j: (j, 0)),
            pl.BlockSpec((rb, 1), lambda i, j: (i, 0)),
            pl.BlockSpec(memory_space=pltpu.SMEM),
        ],
        out_specs=pl.BlockSpec(memory_space=pltpu.SMEM),
        out_shape=jax.ShapeDtypeStruct((1, 1), jnp.float32),
        scratch_shapes=[
            pltpu.VMEM((rb, 1), jnp.float32),
            pltpu.VMEM((rb, 1), jnp.float32),
            pltpu.SMEM((1, 1), jnp.float32),
        ],
    )(temp, x, x, tl, t0)
    return loss[0, 0]


# speedup vs baseline: 41.7701x; 1.4685x over previous
import functools

import jax
import jax.numpy as jnp
from jax.experimental import pallas as pl
from jax.experimental.pallas import tpu as pltpu

_MOMENTUM = 0.99
_REGULAR = 0.1
_T0_SCALE = 1.0 - _MOMENTUM


def _tl_kernel(o1_ref, o2_ref, tl_ref, t0_ref):
    p = jnp.sum(o1_ref[:, :] * o2_ref[:, :], axis=1, keepdims=True)
    p = jnp.clip(p, -1.0, 1.0)
    b = p.shape[0]
    tl_ref[0:b, :] = p
    tl_ref[b:2 * b, :] = p
    t0_ref[0, 0] = _T0_SCALE * jnp.mean(p)


def _loss_kernel(temp_ref, xr_ref, xc_ref, tl_ref, t0_ref, out_ref,
                 sacc_ref, reacc_ref, lacc_ref, *, nbi, nbj, n):
    i = pl.program_id(0)
    j = pl.program_id(1)
    rb = xr_ref.shape[0]
    cb = xc_ref.shape[0]
    t0 = t0_ref[0, 0]
    inv_t = 1.0 / temp_ref[0, 0]

    @pl.when(jnp.logical_and(i == 0, j == 0))
    def _():
        lacc_ref[0, 0] = 0.0

    @pl.when(j == 0)
    def _():
        sacc_ref[:, :] = jnp.zeros_like(sacc_ref)
        reacc_ref[:, :] = jnp.zeros_like(reacc_ref)

    s = jax.lax.dot_general(
        xr_ref[:, :], xc_ref[:, :], (((1,), (1,)), ((), ())),
        preferred_element_type=jnp.float32)
    v = jnp.clip(s, -1.0, 1.0)
    tl = tl_ref[:, :]
    m = v > tl
    w = v + t0
    mod = jnp.where(m, v * w, v)
    rwc = jnp.logical_and(m, w > 1.0)

    @pl.when(i == j)
    def _():
        diag = jax.lax.broadcasted_iota(
            jnp.int32, (rb, cb), 0) == jax.lax.broadcasted_iota(
                jnp.int32, (rb, cb), 1)
        e = jnp.where(diag, 0.0, jnp.exp(mod * inv_t))
        sacc_ref[:, :] += jnp.sum(e, axis=1, keepdims=True)
        rw = jnp.where(jnp.logical_and(rwc, jnp.logical_not(diag)), w, 0.0)
        reacc_ref[:, :] += jnp.sum(rw * rw, axis=1, keepdims=True)

    @pl.when(i != j)
    def _():
        e = jnp.exp(mod * inv_t)
        sacc_ref[:, :] += jnp.sum(e, axis=1, keepdims=True)
        rw = jnp.where(rwc, w, 0.0)
        reacc_ref[:, :] += jnp.sum(rw * rw, axis=1, keepdims=True)

    @pl.when(j == nbj - 1)
    def _():
        ce_part = jnp.sum(jnp.log(sacc_ref[:, :]) - tl_ref[:, :] * inv_t)
        lacc_ref[0, 0] += (ce_part / n
                           + (0.5 * _REGULAR) * jnp.sum(reacc_ref[:, :]))

        @pl.when(i == nbi - 1)
        def _():
            out_ref[0, 0] = lacc_ref[0, 0]


def kernel(out_1, out_2, batch_size, temperature):
    del batch_size
    b, d = out_1.shape
    n = 2 * b
    x = jnp.concatenate([out_1, out_2], axis=0)
    temp = jnp.asarray(temperature, jnp.float32).reshape(1, 1)

    tl, t0 = pl.pallas_call(
        _tl_kernel,
        out_shape=(
            jax.ShapeDtypeStruct((n, 1), jnp.float32),
            jax.ShapeDtypeStruct((1, 1), jnp.float32),
        ),
        out_specs=(
            pl.BlockSpec(memory_space=pltpu.VMEM),
            pl.BlockSpec(memory_space=pltpu.SMEM),
        ),
        in_specs=(
            pl.BlockSpec(memory_space=pltpu.VMEM),
            pl.BlockSpec(memory_space=pltpu.VMEM),
        ),
    )(out_1, out_2)

    rb = 1024
    cb = 1024
    nbi = n // rb
    nbj = n // cb

    body = functools.partial(_loss_kernel, nbi=nbi, nbj=nbj, n=float(n))
    loss = pl.pallas_call(
        body,
        grid=(nbi, nbj),
        in_specs=[
            pl.BlockSpec(memory_space=pltpu.SMEM),
            pl.BlockSpec((rb, d), lambda i, j: (i, 0)),
            pl.BlockSpec((cb, d), lambda i, j: (j, 0)),
            pl.BlockSpec((rb, 1), lambda i, j: (i, 0)),
            pl.BlockSpec(memory_space=pltpu.SMEM),
        ],
        out_specs=pl.BlockSpec(memory_space=pltpu.SMEM),
        out_shape=jax.ShapeDtypeStruct((1, 1), jnp.float32),
        scratch_shapes=[
            pltpu.VMEM((rb, 1), jnp.float32),
            pltpu.VMEM((rb, 1), jnp.float32),
            pltpu.SMEM((1, 1), jnp.float32),
        ],
    )(temp, x, x, tl, t0)
    return loss[0, 0]


# speedup vs baseline: 41.9528x; 1.0044x over previous
import functools

import jax
import jax.numpy as jnp
from jax.experimental import pallas as pl
from jax.experimental.pallas import tpu as pltpu

_MOMENTUM = 0.99
_REGULAR = 0.1
_T0_SCALE = 1.0 - _MOMENTUM


def _tl_kernel(o1_ref, o2_ref, tl_ref, t0_ref):
    p = jnp.sum(o1_ref[:, :] * o2_ref[:, :], axis=1, keepdims=True)
    p = jnp.clip(p, -1.0, 1.0)
    b = p.shape[0]
    tl_ref[0:b, :] = p
    tl_ref[b:2 * b, :] = p
    t0_ref[0, 0] = _T0_SCALE * jnp.mean(p)


_LOG2E = 1.4426950408889634


def _loss_kernel(temp_ref, xr_ref, xc_ref, tl_ref, t0_ref, out_ref,
                 sacc_ref, reacc_ref, *, nbi, nbj, n):
    i = pl.program_id(0)
    j = pl.program_id(1)
    rb = xr_ref.shape[0]
    cb = xc_ref.shape[0]
    t0 = t0_ref[0, 0]
    inv_t = 1.0 / temp_ref[0, 0]
    c2 = inv_t * _LOG2E

    @pl.when(j == 0)
    def _():
        sacc_ref[:, :] = jnp.zeros_like(sacc_ref)
        reacc_ref[:, :] = jnp.zeros_like(reacc_ref)

    s = jax.lax.dot_general(
        xr_ref[:, :], xc_ref[:, :], (((1,), (1,)), ((), ())),
        preferred_element_type=jnp.float32)
    v = jnp.clip(s, -1.0, 1.0)
    tl = tl_ref[:, :]
    m = v > tl
    w = v + t0
    mod = jnp.where(m, v * w, v)
    rwc = jnp.logical_and(m, w > 1.0)

    @pl.when(i == j)
    def _():
        diag = jax.lax.broadcasted_iota(
            jnp.int32, (rb, cb), 0) == jax.lax.broadcasted_iota(
                jnp.int32, (rb, cb), 1)
        e = jnp.where(diag, 0.0, jnp.exp2(mod * c2))
        sacc_ref[:, :] += jnp.sum(e, axis=1, keepdims=True)
        rw = jnp.where(jnp.logical_and(rwc, jnp.logical_not(diag)), w, 0.0)
        reacc_ref[:, :] += jnp.sum(rw * rw, axis=1, keepdims=True)

    @pl.when(i != j)
    def _():
        e = jnp.exp2(mod * c2)
        sacc_ref[:, :] += jnp.sum(e, axis=1, keepdims=True)
        rw = jnp.where(rwc, w, 0.0)
        reacc_ref[:, :] += jnp.sum(rw * rw, axis=1, keepdims=True)

    @pl.when(j == nbj - 1)
    def _():
        ce_part = jnp.sum(jnp.log(sacc_ref[:, :]) - tl_ref[:, :] * inv_t)
        part = ce_part / n + (0.5 * _REGULAR) * jnp.sum(reacc_ref[:, :])
        out_ref[:, :, :] = jnp.full((1, 1, 1), part, jnp.float32)


def kernel(out_1, out_2, batch_size, temperature):
    del batch_size
    b, d = out_1.shape
    n = 2 * b
    x = jnp.concatenate([out_1, out_2], axis=0)
    temp = jnp.asarray(temperature, jnp.float32).reshape(1, 1)

    tl, t0 = pl.pallas_call(
        _tl_kernel,
        out_shape=(
            jax.ShapeDtypeStruct((n, 1), jnp.float32),
            jax.ShapeDtypeStruct((1, 1), jnp.float32),
        ),
        out_specs=(
            pl.BlockSpec(memory_space=pltpu.VMEM),
            pl.BlockSpec(memory_space=pltpu.SMEM),
        ),
        in_specs=(
            pl.BlockSpec(memory_space=pltpu.VMEM),
            pl.BlockSpec(memory_space=pltpu.VMEM),
        ),
    )(out_1, out_2)

    rb = 1024
    cb = 1024
    nbi = n // rb
    nbj = n // cb

    body = functools.partial(_loss_kernel, nbi=nbi, nbj=nbj, n=float(n))
    partials = pl.pallas_call(
        body,
        grid=(nbi, nbj),
        in_specs=[
            pl.BlockSpec(memory_space=pltpu.SMEM),
            pl.BlockSpec((rb, d), lambda i, j: (i, 0)),
            pl.BlockSpec((cb, d), lambda i, j: (j, 0)),
            pl.BlockSpec((rb, 1), lambda i, j: (i, 0)),
            pl.BlockSpec(memory_space=pltpu.SMEM),
        ],
        out_specs=pl.BlockSpec((1, 1, 1), lambda i, j: (i, 0, 0)),
        out_shape=jax.ShapeDtypeStruct((nbi, 1, 1), jnp.float32),
        scratch_shapes=[
            pltpu.VMEM((rb, 1), jnp.float32),
            pltpu.VMEM((rb, 1), jnp.float32),
        ],
        compiler_params=pltpu.CompilerParams(
            dimension_semantics=("parallel", "arbitrary")),
    )(temp, x, x, tl, t0)
    return jnp.sum(partials)


# speedup vs baseline: 50.5594x; 1.2051x over previous
import functools

import jax
import jax.numpy as jnp
from jax.experimental import pallas as pl
from jax.experimental.pallas import tpu as pltpu

_MOMENTUM = 0.99
_REGULAR = 0.1
_T0_SCALE = 1.0 - _MOMENTUM


def _tl_kernel(o1_ref, o2_ref, tl_ref, thr_ref, t0_ref):
    p = jnp.sum(o1_ref[:, :] * o2_ref[:, :], axis=1, keepdims=True)
    p = jnp.clip(p, -1.0, 1.0)
    b = p.shape[0]
    tl_ref[0:b, :] = p
    tl_ref[b:2 * b, :] = p
    t0 = _T0_SCALE * jnp.mean(p)
    t0_ref[0, 0] = t0
    thr = jnp.maximum(p, 1.0 - t0)
    thr_ref[0:b, :] = thr
    thr_ref[b:2 * b, :] = thr


_LOG2E = 1.4426950408889634


def _loss_kernel(temp_ref, xr_ref, xc_ref, tl_ref, thr_ref, t0_ref, out_ref,
                 sacc_ref, reacc_ref, *, nbi, nbj, n):
    i = pl.program_id(0)
    j = pl.program_id(1)
    rb = xr_ref.shape[0]
    cb = xc_ref.shape[0]
    t0 = t0_ref[0, 0]
    inv_t = 1.0 / temp_ref[0, 0]
    c2 = inv_t * _LOG2E

    @pl.when(j == 0)
    def _():
        sacc_ref[:, :] = jnp.zeros_like(sacc_ref)
        reacc_ref[:, :] = jnp.zeros_like(reacc_ref)

    s = jax.lax.dot_general(
        xr_ref[:, :], xc_ref[:, :], (((1,), (1,)), ((), ())),
        preferred_element_type=jnp.float32)
    v = jnp.clip(s, -1.0, 1.0)
    tl = tl_ref[:, :]
    m = v > tl
    w = v + t0
    mod = jnp.where(m, v * w, v)
    rwc = v > thr_ref[:, :]

    on_diag = jnp.logical_and(i * rb < (j + 1) * cb, j * cb < (i + 1) * rb)

    @pl.when(on_diag)
    def _():
        diag = (jax.lax.broadcasted_iota(jnp.int32, (rb, cb), 0)
                + (i * rb - j * cb)) == jax.lax.broadcasted_iota(
                    jnp.int32, (rb, cb), 1)
        e = jnp.where(diag, 0.0, jnp.exp2(mod * c2))
        sacc_ref[:, :] += jnp.sum(e, axis=1, keepdims=True)
        rw = jnp.where(jnp.logical_and(rwc, jnp.logical_not(diag)), w, 0.0)
        reacc_ref[:, :] += jnp.sum(rw * rw, axis=1, keepdims=True)

    @pl.when(jnp.logical_not(on_diag))
    def _():
        e = jnp.exp2(mod * c2)
        sacc_ref[:, :] += jnp.sum(e, axis=1, keepdims=True)
        rw = jnp.where(rwc, w, 0.0)
        reacc_ref[:, :] += jnp.sum(rw * rw, axis=1, keepdims=True)

    @pl.when(j == nbj - 1)
    def _():
        ce_part = jnp.sum(jnp.log(sacc_ref[:, :]) - tl_ref[:, :] * inv_t)
        part = ce_part / n + (0.5 * _REGULAR) * jnp.sum(reacc_ref[:, :])
        out_ref[:, :, :] = jnp.full((1, 1, 1), part, jnp.float32)


def kernel(out_1, out_2, batch_size, temperature):
    del batch_size
    b, d = out_1.shape
    n = 2 * b
    x = jnp.concatenate([out_1, out_2], axis=0)
    temp = jnp.asarray(temperature, jnp.float32).reshape(1, 1)

    tl, thr, t0 = pl.pallas_call(
        _tl_kernel,
        out_shape=(
            jax.ShapeDtypeStruct((n, 1), jnp.float32),
            jax.ShapeDtypeStruct((n, 1), jnp.float32),
            jax.ShapeDtypeStruct((1, 1), jnp.float32),
        ),
        out_specs=(
            pl.BlockSpec(memory_space=pltpu.VMEM),
            pl.BlockSpec(memory_space=pltpu.VMEM),
            pl.BlockSpec(memory_space=pltpu.SMEM),
        ),
        in_specs=(
            pl.BlockSpec(memory_space=pltpu.VMEM),
            pl.BlockSpec(memory_space=pltpu.VMEM),
        ),
    )(out_1, out_2)

    rb = 1024
    cb = 2048
    nbi = n // rb
    nbj = n // cb

    body = functools.partial(_loss_kernel, nbi=nbi, nbj=nbj, n=float(n))
    partials = pl.pallas_call(
        body,
        grid=(nbi, nbj),
        in_specs=[
            pl.BlockSpec(memory_space=pltpu.SMEM),
            pl.BlockSpec((rb, d), lambda i, j: (i, 0)),
            pl.BlockSpec((cb, d), lambda i, j: (j, 0)),
            pl.BlockSpec((rb, 1), lambda i, j: (i, 0)),
            pl.BlockSpec((rb, 1), lambda i, j: (i, 0)),
            pl.BlockSpec(memory_space=pltpu.SMEM),
        ],
        out_specs=pl.BlockSpec((1, 1, 1), lambda i, j: (i, 0, 0)),
        out_shape=jax.ShapeDtypeStruct((nbi, 1, 1), jnp.float32),
        scratch_shapes=[
            pltpu.VMEM((rb, 1), jnp.float32),
            pltpu.VMEM((rb, 1), jnp.float32),
        ],
        compiler_params=pltpu.CompilerParams(
            dimension_semantics=("parallel", "arbitrary")),
    )(temp, x, x, tl, thr, t0)
    return jnp.sum(partials)
